# baseline (speedup 1.0000x reference)
import functools

import jax
import jax.numpy as jnp
from jax import lax
from jax.experimental import pallas as pl
from jax.experimental.pallas import tpu as pltpu, tpu_sc as plsc

B = 16384
Y_DIM = 64

_info = plsc.get_sparse_core_info()
_NC = _info.num_cores
_NS = _info.num_subcores
_L = _info.num_lanes
_NW = _NC * _NS
_B_PER_W = B // _NW
_CHUNK = 128
_N_CHUNKS = _B_PER_W // _CHUNK

_mesh = plsc.VectorSubcoreMesh(core_axis_name="c", subcore_axis_name="s")


@functools.partial(
    pl.kernel,
    mesh=_mesh,
    out_type=jax.ShapeDtypeStruct((B // 2, 2 * Y_DIM), jnp.float32),
    scratch_types=[
        pltpu.VMEM((_B_PER_W,), jnp.int32),
        pltpu.VMEM((_B_PER_W, Y_DIM), jnp.float32),
        pltpu.VMEM((_B_PER_W // 2, 2 * Y_DIM), jnp.float32),
        pltpu.SemaphoreType.DMA,
        pltpu.SemaphoreType.DMA,
    ],
    compiler_params=pltpu.CompilerParams(use_tc_tiling_on_sc=False),
)
def _gather_kernel(idx_hbm, table_hbm, out_hbm, idx_v, rows_v, rows_c, gsem, wsem):
    wid = lax.axis_index("s") * _NC + lax.axis_index("c")
    base = wid * _B_PER_W
    pltpu.sync_copy(idx_hbm.at[pl.ds(base, _B_PER_W)], idx_v)
    gathers = [
        pltpu.async_copy(
            table_hbm.at[idx_v.at[pl.ds(c * _CHUNK, _CHUNK)]],
            rows_v.at[pl.ds(c * _CHUNK, _CHUNK)],
            gsem,
        )
        for c in range(_N_CHUNKS)
    ]
    writes = []
    for c in range(_N_CHUNKS):
        gathers[c].wait()

        def _compact(j, _, c=c):
            r = c * _CHUNK + 2 * j
            for h in range(2):
                for i in range(Y_DIM // _L):
                    rows_c[c * (_CHUNK // 2) + j, pl.ds(h * Y_DIM + i * _L, _L)] = (
                        rows_v[r + h, pl.ds(i * _L, _L)]
                    )
            return 0

        lax.fori_loop(0, _CHUNK // 2, _compact, 0)
        if c == 0:

            @pl.when(wid == 0)
            def _zero_row0():
                for i in range(Y_DIM // _L):
                    rows_c[0, pl.ds(i * _L, _L)] = jnp.zeros((_L,), jnp.float32)

        writes.append(
            pltpu.async_copy(
                rows_c.at[pl.ds(c * (_CHUNK // 2), _CHUNK // 2)],
                out_hbm.at[pl.ds(base // 2 + c * (_CHUNK // 2), _CHUNK // 2)],
                wsem,
            )
        )
    for w in writes:
        w.wait()


def kernel(b, W_loc):
    idx = b.reshape(-1)
    return _gather_kernel(idx, W_loc).reshape(B, Y_DIM)


# speedup vs baseline: 1.1163x; 1.1163x over previous
import functools

import jax
import jax.numpy as jnp
from jax import lax
from jax.experimental import pallas as pl
from jax.experimental.pallas import tpu as pltpu, tpu_sc as plsc

B = 16384
Y_DIM = 64
PAD_DIM = 128

_info = plsc.get_sparse_core_info()
_NC = _info.num_cores
_NS = _info.num_subcores
_L = _info.num_lanes
_NW = _NC * _NS
_B_PER_W = B // _NW
_CHUNK = 128
_N_CHUNKS = _B_PER_W // _CHUNK

_mesh = plsc.VectorSubcoreMesh(core_axis_name="c", subcore_axis_name="s")


@functools.partial(
    pl.kernel,
    mesh=_mesh,
    out_type=jax.ShapeDtypeStruct((B, PAD_DIM), jnp.float32),
    scratch_types=[
        pltpu.VMEM((_B_PER_W,), jnp.int32),
        pltpu.VMEM((_B_PER_W, Y_DIM), jnp.float32),
        pltpu.VMEM((_B_PER_W, PAD_DIM), jnp.float32),
        pltpu.SemaphoreType.DMA,
        pltpu.SemaphoreType.DMA,
    ],
    compiler_params=pltpu.CompilerParams(use_tc_tiling_on_sc=False),
)
def _gather_kernel(idx_hbm, table_hbm, out_hbm, idx_v, rows_v, rows_p, gsem, wsem):
    wid = lax.axis_index("s") * _NC + lax.axis_index("c")
    base = wid * _B_PER_W
    pltpu.sync_copy(idx_hbm.at[pl.ds(base, _B_PER_W)], idx_v)
    gathers = [
        pltpu.async_copy(
            table_hbm.at[idx_v.at[pl.ds(c * _CHUNK, _CHUNK)]],
            rows_v.at[pl.ds(c * _CHUNK, _CHUNK)],
            gsem,
        )
        for c in range(_N_CHUNKS)
    ]
    writes = []
    for c in range(_N_CHUNKS):
        gathers[c].wait()

        def _spread(r, _, c=c):
            row = c * _CHUNK + r
            for i in range(Y_DIM // _L):
                rows_p[row, pl.ds(i * _L, _L)] = rows_v[row, pl.ds(i * _L, _L)]
            return 0

        lax.fori_loop(0, _CHUNK, _spread, 0)
        if c == 0:

            @pl.when(wid == 0)
            def _zero_row0():
                for i in range(Y_DIM // _L):
                    rows_p[0, pl.ds(i * _L, _L)] = jnp.zeros((_L,), jnp.float32)

        writes.append(
            pltpu.async_copy(
                rows_p.at[pl.ds(c * _CHUNK, _CHUNK)],
                out_hbm.at[pl.ds(base + c * _CHUNK, _CHUNK)],
                wsem,
            )
        )
    for w in writes:
        w.wait()


def kernel(b, W_loc):
    idx = b.reshape(-1)
    return _gather_kernel(idx, W_loc)[:, :Y_DIM]


# speedup vs baseline: 1.1541x; 1.0338x over previous
import functools

import jax
import jax.numpy as jnp
from jax import lax
from jax.experimental import pallas as pl
from jax.experimental.pallas import tpu as pltpu, tpu_sc as plsc

B = 16384
Y_DIM = 64
PAD_DIM = 128

_info = plsc.get_sparse_core_info()
_NC = _info.num_cores
_NS = _info.num_subcores
_L = _info.num_lanes
_NW = _NC * _NS
_B_PER_W = B // _NW

_mesh = plsc.VectorSubcoreMesh(core_axis_name="c", subcore_axis_name="s")


@functools.partial(
    pl.kernel,
    mesh=_mesh,
    out_type=jax.ShapeDtypeStruct((B, PAD_DIM), jnp.float32),
    scratch_types=[
        pltpu.VMEM((_B_PER_W,), jnp.int32),
        pltpu.VMEM((_B_PER_W, PAD_DIM), jnp.float32),
        pltpu.SemaphoreType.DMA,
    ],
)
def _gather_kernel(idx_hbm, table_hbm, out_hbm, idx_v, rows_v, sem):
    wid = lax.axis_index("s") * _NC + lax.axis_index("c")
    base = wid * _B_PER_W
    pltpu.sync_copy(idx_hbm.at[pl.ds(base, _B_PER_W)], idx_v)
    pltpu.async_copy(table_hbm.at[idx_v], rows_v, sem).wait()

    @pl.when(wid == 0)
    def _zero_row0():
        for i in range(PAD_DIM // _L):
            rows_v[0, pl.ds(i * _L, _L)] = jnp.zeros((_L,), jnp.float32)

    pltpu.sync_copy(rows_v, out_hbm.at[pl.ds(base, _B_PER_W)])


def kernel(b, W_loc):
    idx = b.reshape(-1)
    table = jnp.pad(W_loc, ((0, 0), (0, PAD_DIM - Y_DIM)))
    out = _gather_kernel(idx, table)
    return out[:, :Y_DIM]
